# baseline (speedup 1.0000x reference)
import functools
import math

import jax
import jax.numpy as jnp
from jax import lax
from jax.experimental import pallas as pl
from jax.experimental.pallas import tpu as pltpu
from jax.experimental.pallas import tpu_sc as plsc

_NUM_CORES = 2
_NUM_SUBCORES = 16
_NW = _NUM_CORES * _NUM_SUBCORES
_CHUNK = 128


def _table_body(tab_ref, w1_ref, b1_ref, w2_ref, b2_ref, out_ref):
    nb = tab_ref.shape[0]
    h = w2_ref.shape[0]
    hour = lax.broadcasted_iota(jnp.int32, (nb, h), 0).astype(jnp.float32)
    ang = 2.0 * math.pi * hour / 24.0
    s = jnp.sin(ang)
    c = jnp.cos(ang)
    hidden = jnp.maximum(s * w1_ref[0:1, :] + c * w1_ref[1:2, :] + b1_ref[:], 0.0)
    cont = jnp.dot(hidden, w2_ref[:], preferred_element_type=jnp.float32) + b2_ref[:]
    out_ref[:, : tab_ref.shape[1]] = tab_ref[:]
    out_ref[:, tab_ref.shape[1] :] = cont


def _build_table(hour_table, W1, b1, W2, b2):
    nb, e = hour_table.shape
    h = W2.shape[0]
    return pl.pallas_call(
        _table_body,
        out_shape=jax.ShapeDtypeStruct((nb, e + h), jnp.float32),
    )(hour_table, W1, b1.reshape(1, h), W2, b2.reshape(1, h))


def _make_gather(n, d, nb):
    assert n % (_NW * _CHUNK) == 0
    bpw = n // _NW
    nchunk = bpw // _CHUNK
    assert nchunk % 2 == 0
    cd = _CHUNK * d
    mesh = plsc.VectorSubcoreMesh(core_axis_name="c", subcore_axis_name="s")

    @functools.partial(
        pl.kernel,
        mesh=mesh,
        compiler_params=pltpu.CompilerParams(
            use_tc_tiling_on_sc=False, needs_layout_passes=False
        ),
        out_type=jax.ShapeDtypeStruct((n * d,), jnp.float32),
        scratch_types=[
            pltpu.VMEM((bpw,), jnp.int32),
            pltpu.VMEM((d * nb,), jnp.float32),
            pltpu.VMEM((cd,), jnp.float32),
            pltpu.VMEM((cd,), jnp.float32),
            pltpu.SemaphoreType.DMA,
            pltpu.SemaphoreType.DMA,
        ],
    )
    def gather_kernel(table_hbm, idx_hbm, out_hbm, idx_v, tab_v, buf0, buf1, w0, w1):
        wid = lax.axis_index("s") * _NUM_CORES + lax.axis_index("c")
        base = wid * bpw
        pltpu.sync_copy(table_hbm, tab_v)
        pltpu.sync_copy(idx_hbm.at[pl.ds(base, bpw)], idx_v)
        bufs = (buf0, buf1)
        wsems = (w0, w1)
        iota = lax.broadcasted_iota(jnp.int32, (16,), 0)
        iota_d = iota * d

        def compute_chunk(c, buf):
            row0 = c * _CHUNK

            def group(g, _):
                iv = idx_v[pl.ds(row0 + g * 16, 16)]
                rbase = iota_d + g * (16 * d)
                for k in range(d):
                    vals = plsc.load_gather(tab_v, [iv + k * nb])
                    plsc.store_scatter(buf, [rbase + k], vals)
                return 0

            lax.fori_loop(0, _CHUNK // 16, group, 0)

        def wb_start(c, b):
            pltpu.async_copy(
                bufs[b], out_hbm.at[pl.ds((base + c * _CHUNK) * d, cd)], wsems[b]
            )

        def wb_wait(b):
            pltpu.make_async_copy(bufs[b], out_hbm.at[pl.ds(0, cd)], wsems[b]).wait()

        for b in range(2):
            compute_chunk(b, bufs[b])
            wb_start(b, b)

        def body(p, _):
            for b in range(2):
                c = 2 * p + b
                wb_wait(b)
                compute_chunk(c, bufs[b])
                wb_start(c, b)
            return 0

        lax.fori_loop(1, nchunk // 2, body, 0)

        for b in range(2):
            wb_wait(b)

    return gather_kernel


def kernel(hours, hour_table, W1, b1, W2, b2):
    table = _build_table(hour_table, W1, b1, W2, b2)
    nb, d = table.shape
    table_t = table.T.reshape(-1)
    flat = hours.reshape(-1)
    n = flat.shape[0]
    out = _make_gather(n, d, nb)(table_t, flat)
    return out.reshape(*hours.shape, d)


# speedup vs baseline: 1.0018x; 1.0018x over previous
import functools
import math

import jax
import jax.numpy as jnp
from jax import lax
from jax.experimental import pallas as pl
from jax.experimental.pallas import tpu as pltpu
from jax.experimental.pallas import tpu_sc as plsc

_NUM_CORES = 2
_NUM_SUBCORES = 16
_NW = _NUM_CORES * _NUM_SUBCORES
_CHUNK = 256


def _table_body(tab_ref, w1_ref, b1_ref, w2_ref, b2_ref, out_ref):
    nb = tab_ref.shape[0]
    h = w2_ref.shape[0]
    hour = lax.broadcasted_iota(jnp.int32, (nb, h), 0).astype(jnp.float32)
    ang = 2.0 * math.pi * hour / 24.0
    s = jnp.sin(ang)
    c = jnp.cos(ang)
    hidden = jnp.maximum(s * w1_ref[0:1, :] + c * w1_ref[1:2, :] + b1_ref[:], 0.0)
    cont = jnp.dot(hidden, w2_ref[:], preferred_element_type=jnp.float32) + b2_ref[:]
    out_ref[:, : tab_ref.shape[1]] = tab_ref[:]
    out_ref[:, tab_ref.shape[1] :] = cont


def _build_table(hour_table, W1, b1, W2, b2):
    nb, e = hour_table.shape
    h = W2.shape[0]
    return pl.pallas_call(
        _table_body,
        out_shape=jax.ShapeDtypeStruct((nb, e + h), jnp.float32),
    )(hour_table, W1, b1.reshape(1, h), W2, b2.reshape(1, h))


def _make_gather(n, d, nb):
    assert n % (_NW * _CHUNK) == 0
    bpw = n // _NW
    nchunk = bpw // _CHUNK
    assert nchunk >= 3
    cd = _CHUNK * d
    mesh = plsc.VectorSubcoreMesh(core_axis_name="c", subcore_axis_name="s")

    @functools.partial(
        pl.kernel,
        mesh=mesh,
        compiler_params=pltpu.CompilerParams(
            use_tc_tiling_on_sc=False, needs_layout_passes=False
        ),
        out_type=jax.ShapeDtypeStruct((n * d,), jnp.float32),
        scratch_types=[
            pltpu.VMEM((bpw,), jnp.int32),
            pltpu.VMEM((d * nb,), jnp.float32),
            pltpu.VMEM((cd,), jnp.float32),
            pltpu.VMEM((cd,), jnp.float32),
            pltpu.SemaphoreType.DMA,
            pltpu.SemaphoreType.DMA,
        ],
    )
    def gather_kernel(table_hbm, idx_hbm, out_hbm, idx_v, tab_v, buf0, buf1, w0, w1):
        wid = lax.axis_index("s") * _NUM_CORES + lax.axis_index("c")
        base = wid * bpw
        pltpu.sync_copy(table_hbm, tab_v)
        pltpu.sync_copy(idx_hbm.at[pl.ds(base, bpw)], idx_v)
        bufs = (buf0, buf1)
        wsems = (w0, w1)
        iota = lax.broadcasted_iota(jnp.int32, (16,), 0)
        iota_d = iota * d

        def compute_chunk(c, buf):
            row0 = c * _CHUNK

            def group(g, _):
                iv = idx_v[pl.ds(row0 + g * 16, 16)]
                rbase = iota_d + g * (16 * d)
                for k in range(d):
                    vals = plsc.load_gather(tab_v, [iv + k * nb])
                    plsc.store_scatter(buf, [rbase + k], vals)
                return 0

            lax.fori_loop(0, _CHUNK // 16, group, 0)

        def wb_start(c, b):
            pltpu.async_copy(
                bufs[b], out_hbm.at[pl.ds((base + c * _CHUNK) * d, cd)], wsems[b]
            )

        def wb_wait(b):
            pltpu.make_async_copy(bufs[b], out_hbm.at[pl.ds(0, cd)], wsems[b]).wait()

        for b in range(2):
            compute_chunk(b, bufs[b])
            wb_start(b, b)

        def body(p, _):
            for b in range(2):
                c = 2 * p + b
                wb_wait(b)
                compute_chunk(c, bufs[b])
                wb_start(c, b)
            return 0

        lax.fori_loop(1, nchunk // 2, body, 0)

        if nchunk % 2:
            c = nchunk - 1
            b = c % 2
            wb_wait(b)
            compute_chunk(c, bufs[b])
            wb_start(c, b)

        for b in range(2):
            wb_wait(b)

    return gather_kernel


def kernel(hours, hour_table, W1, b1, W2, b2):
    table = _build_table(hour_table, W1, b1, W2, b2)
    nb, d = table.shape
    table_t = table.T.reshape(-1)
    flat = hours.reshape(-1)
    n = flat.shape[0]
    out = _make_gather(n, d, nb)(table_t, flat)
    return out.reshape(*hours.shape, d)


# speedup vs baseline: 1.2365x; 1.2344x over previous
import functools
import math

import jax
import jax.numpy as jnp
from jax import lax
from jax.experimental import pallas as pl
from jax.experimental.pallas import tpu as pltpu
from jax.experimental.pallas import tpu_sc as plsc

_NUM_CORES = 2
_NUM_SUBCORES = 16
_NW = _NUM_CORES * _NUM_SUBCORES
_CHUNK = 256


def _table_body(tab_ref, w1_ref, b1_ref, w2_ref, b2_ref, out_ref):
    nb = tab_ref.shape[0]
    h = w2_ref.shape[0]
    hour = lax.broadcasted_iota(jnp.int32, (nb, h), 0).astype(jnp.float32)
    ang = 2.0 * math.pi * hour / 24.0
    s = jnp.sin(ang)
    c = jnp.cos(ang)
    hidden = jnp.maximum(s * w1_ref[0:1, :] + c * w1_ref[1:2, :] + b1_ref[:], 0.0)
    cont = jnp.dot(hidden, w2_ref[:], preferred_element_type=jnp.float32) + b2_ref[:]
    out_ref[:, : tab_ref.shape[1]] = tab_ref[:]
    out_ref[:, tab_ref.shape[1] :] = cont


def _build_table(hour_table, W1, b1, W2, b2):
    nb, e = hour_table.shape
    h = W2.shape[0]
    return pl.pallas_call(
        _table_body,
        out_shape=jax.ShapeDtypeStruct((nb, e + h), jnp.float32),
    )(hour_table, W1, b1.reshape(1, h), W2, b2.reshape(1, h))


def _make_gather(n, d, nb):
    assert n % (_NW * _CHUNK) == 0
    bpw = n // _NW
    nchunk = bpw // _CHUNK
    assert nchunk >= 3
    cd = _CHUNK * d
    mesh = plsc.VectorSubcoreMesh(core_axis_name="c", subcore_axis_name="s")

    @functools.partial(
        pl.kernel,
        mesh=mesh,
        compiler_params=pltpu.CompilerParams(
            use_tc_tiling_on_sc=False, needs_layout_passes=False
        ),
        out_type=jax.ShapeDtypeStruct((n * d,), jnp.float32),
        scratch_types=[
            pltpu.VMEM((bpw,), jnp.int32),
            pltpu.VMEM((d * nb,), jnp.float32),
            pltpu.VMEM((cd,), jnp.float32),
            pltpu.VMEM((cd,), jnp.float32),
            pltpu.SemaphoreType.DMA,
            pltpu.SemaphoreType.DMA,
        ],
    )
    def gather_kernel(table_hbm, idx_hbm, out_hbm, idx_v, tab_v, buf0, buf1, w0, w1):
        wid = lax.axis_index("s") * _NUM_CORES + lax.axis_index("c")
        base = wid * bpw
        pltpu.sync_copy(table_hbm, tab_v)
        pltpu.sync_copy(idx_hbm.at[pl.ds(base, bpw)], idx_v)
        bufs = (buf0, buf1)
        wsems = (w0, w1)
        iota = lax.broadcasted_iota(jnp.int32, (16,), 0)
        iota_d = iota * d

        def compute_chunk(c, buf):
            row0 = c * _CHUNK

            def group(g, _):
                iv = idx_v[pl.ds(row0 + g * 16, 16)]
                rbase = iota_d + g * (16 * d)

                @plsc.parallel_loop(0, d, unroll=16)
                def _k(k):
                    vals = plsc.load_gather(tab_v, [iv + k * nb])
                    plsc.store_scatter(buf, [rbase + k], vals)

                return 0

            lax.fori_loop(0, _CHUNK // 16, group, 0)

        def wb_start(c, b):
            pltpu.async_copy(
                bufs[b], out_hbm.at[pl.ds((base + c * _CHUNK) * d, cd)], wsems[b]
            )

        def wb_wait(b):
            pltpu.make_async_copy(bufs[b], out_hbm.at[pl.ds(0, cd)], wsems[b]).wait()

        for b in range(2):
            compute_chunk(b, bufs[b])
            wb_start(b, b)

        def body(p, _):
            for b in range(2):
                c = 2 * p + b
                wb_wait(b)
                compute_chunk(c, bufs[b])
                wb_start(c, b)
            return 0

        lax.fori_loop(1, nchunk // 2, body, 0)

        if nchunk % 2:
            c = nchunk - 1
            b = c % 2
            wb_wait(b)
            compute_chunk(c, bufs[b])
            wb_start(c, b)

        for b in range(2):
            wb_wait(b)

    return gather_kernel


def kernel(hours, hour_table, W1, b1, W2, b2):
    table = _build_table(hour_table, W1, b1, W2, b2)
    nb, d = table.shape
    table_t = table.T.reshape(-1)
    flat = hours.reshape(-1)
    n = flat.shape[0]
    out = _make_gather(n, d, nb)(table_t, flat)
    return out.reshape(*hours.shape, d)


# speedup vs baseline: 2.5521x; 2.0639x over previous
import functools
import math

import jax
import jax.numpy as jnp
from jax import lax
from jax.experimental import pallas as pl
from jax.experimental.pallas import tpu as pltpu
from jax.experimental.pallas import tpu_sc as plsc

_NUM_CORES = 2
_NUM_SUBCORES = 16
_NW = _NUM_CORES * _NUM_SUBCORES
_CHUNK = 256


def _table_body(tab_ref, w1_ref, b1_ref, w2_ref, b2_ref, out_ref):
    nb = tab_ref.shape[0]
    h = w2_ref.shape[0]
    hour = lax.broadcasted_iota(jnp.int32, (nb, h), 0).astype(jnp.float32)
    ang = 2.0 * math.pi * hour / 24.0
    s = jnp.sin(ang)
    c = jnp.cos(ang)
    hidden = jnp.maximum(s * w1_ref[0:1, :] + c * w1_ref[1:2, :] + b1_ref[:], 0.0)
    cont = jnp.dot(hidden, w2_ref[:], preferred_element_type=jnp.float32) + b2_ref[:]
    out_ref[:, : tab_ref.shape[1]] = tab_ref[:]
    out_ref[:, tab_ref.shape[1] :] = cont


def _build_table(hour_table, W1, b1, W2, b2):
    nb, e = hour_table.shape
    h = W2.shape[0]
    return pl.pallas_call(
        _table_body,
        out_shape=jax.ShapeDtypeStruct((nb, e + h), jnp.float32),
    )(hour_table, W1, b1.reshape(1, h), W2, b2.reshape(1, h))


def _make_gather(n, d, nb):
    assert n % (_NW * _CHUNK) == 0
    bpw = n // _NW
    nchunk = bpw // _CHUNK
    assert nchunk >= 3
    cd = _CHUNK * d
    mesh = plsc.VectorSubcoreMesh(core_axis_name="c", subcore_axis_name="s")

    @functools.partial(
        pl.kernel,
        mesh=mesh,
        compiler_params=pltpu.CompilerParams(
            use_tc_tiling_on_sc=False, needs_layout_passes=False
        ),
        out_type=jax.ShapeDtypeStruct((n * d,), jnp.float32),
        scratch_types=[
            pltpu.VMEM((bpw,), jnp.int32),
            pltpu.VMEM((d * nb,), jnp.float32),
            pltpu.VMEM((cd,), jnp.float32),
            pltpu.VMEM((cd,), jnp.float32),
            pltpu.SemaphoreType.DMA,
            pltpu.SemaphoreType.DMA,
        ],
    )
    def gather_kernel(table_hbm, idx_hbm, out_hbm, idx_v, tab_v, buf0, buf1, w0, w1):
        wid = lax.axis_index("s") * _NUM_CORES + lax.axis_index("c")
        base = wid * bpw
        pltpu.sync_copy(table_hbm, tab_v)
        pltpu.sync_copy(idx_hbm.at[pl.ds(base, bpw)], idx_v)
        bufs = (buf0, buf1)
        wsems = (w0, w1)
        iota = lax.broadcasted_iota(jnp.int32, (16,), 0)
        iota_d = iota * d

        def compute_chunk(c, buf):
            row0 = c * _CHUNK

            def group(g, _):
                iv = idx_v[pl.ds(row0 + g * 16, 16)]
                ivs = iv * d
                rbase = iota_d + g * (16 * d)

                @plsc.parallel_loop(0, d, unroll=16)
                def _k(k):
                    cvec = jnp.bitwise_xor(iota, k)
                    vals = plsc.load_gather(tab_v, [ivs + cvec])
                    plsc.store_scatter(buf, [rbase + cvec], vals)

                return 0

            lax.fori_loop(0, _CHUNK // 16, group, 0)

        def wb_start(c, b):
            pltpu.async_copy(
                bufs[b], out_hbm.at[pl.ds((base + c * _CHUNK) * d, cd)], wsems[b]
            )

        def wb_wait(b):
            pltpu.make_async_copy(bufs[b], out_hbm.at[pl.ds(0, cd)], wsems[b]).wait()

        for b in range(2):
            compute_chunk(b, bufs[b])
            wb_start(b, b)

        def body(p, _):
            for b in range(2):
                c = 2 * p + b
                wb_wait(b)
                compute_chunk(c, bufs[b])
                wb_start(c, b)
            return 0

        lax.fori_loop(1, nchunk // 2, body, 0)

        if nchunk % 2:
            c = nchunk - 1
            b = c % 2
            wb_wait(b)
            compute_chunk(c, bufs[b])
            wb_start(c, b)

        for b in range(2):
            wb_wait(b)

    return gather_kernel


def kernel(hours, hour_table, W1, b1, W2, b2):
    table = _build_table(hour_table, W1, b1, W2, b2)
    nb, d = table.shape
    table_t = table.reshape(-1)
    flat = hours.reshape(-1)
    n = flat.shape[0]
    out = _make_gather(n, d, nb)(table_t, flat)
    return out.reshape(*hours.shape, d)
